# baseline (speedup 1.0000x reference)
import functools

import jax
import jax.numpy as jnp
from jax import lax
from jax.experimental import pallas as pl
from jax.experimental.pallas import tpu as pltpu, tpu_sc as plsc

VOCAB = 10000
VPAD = 10240
EMBED = 16
SEQ = 20
HID = 16

_info = plsc.get_sparse_core_info()
_NC, _NS, _LANES = _info.num_cores, _info.num_subcores, _info.num_lanes
_NW = _NC * _NS


def _t2_body(e8_ref, w1_ref, out_ref):
    e8 = e8_ref[...]
    r = lax.broadcasted_iota(jnp.int32, (128, 128), 0)
    c = lax.broadcasted_iota(jnp.int32, (128, 128), 1)
    diag = (r // HID) == (c // HID)
    vp8 = VPAD // 8
    for l in range(SEQ):
        bd = jnp.where(diag, jnp.tile(w1_ref[l], (8, 8)), 0.0)
        out_ref[pl.ds(l * vp8, vp8), :] = jnp.dot(
            e8, bd, preferred_element_type=jnp.float32)


def _build_t2(e8, w1_3d):
    return pl.pallas_call(
        _t2_body,
        grid=(1,),
        in_specs=[
            pl.BlockSpec((VPAD // 8, 128), lambda i: (0, 0)),
            pl.BlockSpec((SEQ, EMBED, HID), lambda i: (0, 0, 0)),
        ],
        out_specs=pl.BlockSpec((SEQ * VPAD // 8, 128), lambda i: (0, 0)),
        out_shape=jax.ShapeDtypeStruct((SEQ * VPAD // 8, 128), jnp.float32),
    )(e8, w1_3d)


def _make_sc_kernel(batch):
    rows_w = batch // _NW
    idx_w = rows_w * SEQ
    rows_c = 64
    idx_c = rows_c * SEQ
    g_per_c = idx_c // 128
    n_chunks = rows_w // rows_c
    n_groups = rows_c // 16

    mesh = plsc.VectorSubcoreMesh(core_axis_name="c", subcore_axis_name="s")

    @functools.partial(
        pl.kernel,
        mesh=mesh,
        compiler_params=pltpu.CompilerParams(use_tc_tiling_on_sc=False),
        out_type=jax.ShapeDtypeStruct((batch // 8, 128), jnp.float32),
        scratch_types=[
            pltpu.VMEM((idx_w,), jnp.int32),
            pltpu.VMEM((idx_w,), jnp.int32),
            pltpu.VMEM((4 * idx_c, HID), jnp.float32),
            pltpu.VMEM((rows_w // 8, 128), jnp.float32),
            pltpu.SemaphoreType.DMA,
        ],
    )
    def sc_kernel(xf_hbm, t2_hbm, out_hbm, raw_v, idx_v, buf_v, o_v, sem):
        wid = lax.axis_index("s") * _NC + lax.axis_index("c")
        row0 = wid * rows_w

        pltpu.sync_copy(xf_hbm.at[pl.ds(row0 * SEQ, idx_w)], raw_v)
        lane = lax.iota(jnp.int32, 16)
        zero = jnp.zeros((16,), jnp.float32)

        offs = [((lane + p * 16) % SEQ) * VPAD for p in range(5)]

        def prep(i5, _):
            for p in range(5):
                i = i5 * 5 + p
                idx_v[pl.ds(i * 16, 16)] = (
                    raw_v[pl.ds(i * 16, 16)] + offs[p])
            return 0
        lax.fori_loop(0, idx_w // 80, prep, 0)

        def fire(c, slot):
            for j in range(g_per_c):
                pltpu.async_copy(
                    t2_hbm.at[idx_v.at[pl.ds((c * g_per_c + j) * 128, 128)]],
                    buf_v.at[pl.ds(slot * idx_c + j * 128, 128)],
                    sem)

        fire(0, 0)
        fire(1, 1)
        fire(2, 2)

        def chunk(c, _):
            slot = c % 4
            pltpu.make_async_copy(
                t2_hbm.at[pl.ds(0, idx_c)],
                buf_v.at[pl.ds(slot * idx_c, idx_c)],
                sem).wait()

            @pl.when(c + 3 < n_chunks)
            def _():
                fire(c + 3, (c + 3) % 4)

            def group(g, _):
                for rr in range(16):
                    acc = zero
                    for l in range(SEQ):
                        acc = acc + buf_v[slot * idx_c
                                          + g * (16 * SEQ) + rr * SEQ + l, :]
                    o_v[c * 8 + g * 2 + rr // 8,
                        pl.ds((rr % 8) * HID, HID)] = acc
                return 0
            lax.fori_loop(0, n_groups, group, 0)
            return 0
        lax.fori_loop(0, n_chunks, chunk, 0)

        pltpu.sync_copy(o_v, out_hbm.at[pl.ds(row0 // 8, rows_w // 8)])

    return sc_kernel


def _head_body(acc_ref, b1_ref, w2_ref, b2_ref, out_ref):
    b1t = jnp.tile(b1_ref[...], (1, 8))
    h = jnp.maximum(acc_ref[...] + b1t, 0.0)
    w2t = jnp.tile(w2_ref[...], (8, 8))
    r = lax.broadcasted_iota(jnp.int32, (128, 8), 0)
    c = lax.broadcasted_iota(jnp.int32, (128, 8), 1)
    w2bd = jnp.where((r // HID) == c, w2t, 0.0)
    z = jnp.dot(h, w2bd, preferred_element_type=jnp.float32)
    out_ref[...] = 1.0 / (1.0 + jnp.exp(-(z + b2_ref[...])))


def _apply_head(acc8, b1_, w2_, b2_):
    n = acc8.shape[0]
    return pl.pallas_call(
        _head_body,
        grid=(1,),
        in_specs=[
            pl.BlockSpec((n, 128), lambda i: (0, 0)),
            pl.BlockSpec((1, HID), lambda i: (0, 0)),
            pl.BlockSpec((HID, 1), lambda i: (0, 0)),
            pl.BlockSpec((1, 1), lambda i: (0, 0)),
        ],
        out_specs=pl.BlockSpec((n, 8), lambda i: (0, 0)),
        out_shape=jax.ShapeDtypeStruct((n, 8), jnp.float32),
    )(acc8, b1_, w2_, b2_)


def kernel(x, emb, W1, b1, W2, b2):
    batch = x.shape[0]
    f32 = jnp.float32
    e8 = jnp.pad(emb.astype(f32), ((0, VPAD - VOCAB), (0, 0))
                 ).reshape(VPAD // 8, 128)
    w1_3d = W1.astype(f32).reshape(SEQ, EMBED, HID)
    t2p = _build_t2(e8, w1_3d)

    xf2 = x.astype(jnp.int32).reshape(batch * SEQ // 128, 128)
    acc8 = _make_sc_kernel(batch)(xf2.reshape(-1),
                                  t2p.reshape(SEQ * VPAD, HID))

    out8 = _apply_head(acc8, b1.astype(f32).reshape(1, HID),
                       W2.astype(f32), b2.astype(f32).reshape(1, 1))
    return out8.reshape(batch, 1)


# speedup vs baseline: 1.1169x; 1.1169x over previous
import functools

import jax
import jax.numpy as jnp
from jax import lax
from jax.experimental import pallas as pl
from jax.experimental.pallas import tpu as pltpu, tpu_sc as plsc

VOCAB = 10000
VPAD = 10240
EMBED = 16
SEQ = 20
HID = 16

_info = plsc.get_sparse_core_info()
_NC, _NS, _LANES = _info.num_cores, _info.num_subcores, _info.num_lanes
_NW = _NC * _NS


def _t2_body(e8_ref, w1_ref, out_ref):
    e8 = e8_ref[...]
    r = lax.broadcasted_iota(jnp.int32, (128, 128), 0)
    c = lax.broadcasted_iota(jnp.int32, (128, 128), 1)
    diag = (r // HID) == (c // HID)
    vp8 = VPAD // 8
    for l in range(SEQ):
        bd = jnp.where(diag, jnp.tile(w1_ref[l], (8, 8)), 0.0)
        out_ref[pl.ds(l * vp8, vp8), :] = jnp.dot(
            e8, bd, preferred_element_type=jnp.float32)


def _build_t2(e8, w1_3d):
    return pl.pallas_call(
        _t2_body,
        grid=(1,),
        in_specs=[
            pl.BlockSpec((VPAD // 8, 128), lambda i: (0, 0)),
            pl.BlockSpec((SEQ, EMBED, HID), lambda i: (0, 0, 0)),
        ],
        out_specs=pl.BlockSpec((SEQ * VPAD // 8, 128), lambda i: (0, 0)),
        out_shape=jax.ShapeDtypeStruct((SEQ * VPAD // 8, 128), jnp.float32),
    )(e8, w1_3d)


def _make_flattener(batch):
    rows_w = batch // _NW
    idx_w = rows_w * SEQ

    mesh = plsc.VectorSubcoreMesh(core_axis_name="c", subcore_axis_name="s")

    @functools.partial(
        pl.kernel,
        mesh=mesh,
        out_type=jax.ShapeDtypeStruct((batch * SEQ,), jnp.int32),
        scratch_types=[
            pltpu.VMEM((rows_w, SEQ), jnp.int32),
            pltpu.VMEM((idx_w,), jnp.int32),
        ],
    )
    def flat_kernel(x_hbm, out_hbm, raw_v, o_v):
        wid = lax.axis_index("s") * _NC + lax.axis_index("c")
        row0 = wid * rows_w
        pltpu.sync_copy(x_hbm.at[pl.ds(row0, rows_w)], raw_v)
        lane = lax.iota(jnp.int32, 16)
        off0 = lane * VPAD
        off1 = (lane + 4) * VPAD

        def row(r, _):
            o_v[pl.ds(r * SEQ, 16)] = raw_v[r, pl.ds(0, 16)] + off0
            o_v[pl.ds(r * SEQ + 4, 16)] = raw_v[r, pl.ds(4, 16)] + off1
            return 0
        lax.fori_loop(0, rows_w, row, 0)

        pltpu.sync_copy(o_v, out_hbm.at[pl.ds(row0 * SEQ, idx_w)])

    return flat_kernel


def _make_sc_kernel(batch):
    rows_w = batch // _NW
    idx_w = rows_w * SEQ
    rows_c = 64
    idx_c = rows_c * SEQ
    g_per_c = idx_c // 128
    n_chunks = rows_w // rows_c
    n_groups = rows_c // 16

    mesh = plsc.VectorSubcoreMesh(core_axis_name="c", subcore_axis_name="s")

    @functools.partial(
        pl.kernel,
        mesh=mesh,
        compiler_params=pltpu.CompilerParams(use_tc_tiling_on_sc=False),
        out_type=jax.ShapeDtypeStruct((batch // 8, 128), jnp.float32),
        scratch_types=[
            pltpu.VMEM((idx_w,), jnp.int32),
            pltpu.VMEM((4 * idx_c, HID), jnp.float32),
            pltpu.VMEM((rows_w // 8, 128), jnp.float32),
            pltpu.SemaphoreType.DMA,
        ],
    )
    def sc_kernel(xf_hbm, t2_hbm, out_hbm, idx_v, buf_v, o_v, sem):
        wid = lax.axis_index("s") * _NC + lax.axis_index("c")
        row0 = wid * rows_w

        pltpu.sync_copy(xf_hbm.at[pl.ds(row0 * SEQ, idx_w)], idx_v)
        zero = jnp.zeros((16,), jnp.float32)

        def fire(c, slot):
            for j in range(g_per_c):
                pltpu.async_copy(
                    t2_hbm.at[idx_v.at[pl.ds((c * g_per_c + j) * 128, 128)]],
                    buf_v.at[pl.ds(slot * idx_c + j * 128, 128)],
                    sem)

        fire(0, 0)
        fire(1, 1)
        fire(2, 2)

        def chunk(c, _):
            slot = c % 4
            pltpu.make_async_copy(
                t2_hbm.at[pl.ds(0, idx_c)],
                buf_v.at[pl.ds(slot * idx_c, idx_c)],
                sem).wait()

            @pl.when(c + 3 < n_chunks)
            def _():
                fire(c + 3, (c + 3) % 4)

            def group(g, _):
                for rr in range(16):
                    acc = zero
                    for l in range(SEQ):
                        acc = acc + buf_v[slot * idx_c
                                          + g * (16 * SEQ) + rr * SEQ + l, :]
                    o_v[c * 8 + g * 2 + rr // 8,
                        pl.ds((rr % 8) * HID, HID)] = acc
                return 0
            lax.fori_loop(0, n_groups, group, 0)
            return 0
        lax.fori_loop(0, n_chunks, chunk, 0)

        pltpu.sync_copy(o_v, out_hbm.at[pl.ds(row0 // 8, rows_w // 8)])

    return sc_kernel


def _head_body(acc_ref, b1_ref, w2_ref, b2_ref, out_ref):
    b1t = jnp.tile(b1_ref[...], (1, 8))
    h = jnp.maximum(acc_ref[...] + b1t, 0.0)
    w2t = jnp.tile(w2_ref[...], (8, 8))
    r = lax.broadcasted_iota(jnp.int32, (128, 8), 0)
    c = lax.broadcasted_iota(jnp.int32, (128, 8), 1)
    w2bd = jnp.where((r // HID) == c, w2t, 0.0)
    z = jnp.dot(h, w2bd, preferred_element_type=jnp.float32)
    out_ref[...] = 1.0 / (1.0 + jnp.exp(-(z + b2_ref[...])))


def _apply_head(acc8, b1_, w2_, b2_):
    n = acc8.shape[0]
    return pl.pallas_call(
        _head_body,
        grid=(1,),
        in_specs=[
            pl.BlockSpec((n, 128), lambda i: (0, 0)),
            pl.BlockSpec((1, HID), lambda i: (0, 0)),
            pl.BlockSpec((HID, 1), lambda i: (0, 0)),
            pl.BlockSpec((1, 1), lambda i: (0, 0)),
        ],
        out_specs=pl.BlockSpec((n, 8), lambda i: (0, 0)),
        out_shape=jax.ShapeDtypeStruct((n, 8), jnp.float32),
    )(acc8, b1_, w2_, b2_)


def kernel(x, emb, W1, b1, W2, b2):
    batch = x.shape[0]
    f32 = jnp.float32
    e8 = jnp.pad(emb.astype(f32), ((0, VPAD - VOCAB), (0, 0))
                 ).reshape(VPAD // 8, 128)
    w1_3d = W1.astype(f32).reshape(SEQ, EMBED, HID)
    t2p = _build_t2(e8, w1_3d)

    idx_flat = _make_flattener(batch)(x.astype(jnp.int32))
    acc8 = _make_sc_kernel(batch)(idx_flat, t2p.reshape(SEQ * VPAD, HID))

    out8 = _apply_head(acc8, b1.astype(f32).reshape(1, HID),
                       W2.astype(f32), b2.astype(f32).reshape(1, 1))
    return out8.reshape(batch, 1)
